# Initial kernel scaffold; baseline (speedup 1.0000x reference)
import jax
import jax.numpy as jnp
from jax.experimental import pallas as pl


def kernel(x, x_entity, x_negation, target, text_inputs, use_elmo, W, W_entity, W_negation):
    raise NotImplementedError("write your pallas kernel here")


# speedup vs baseline: 1.7262x; 1.7262x over previous
import functools
import jax
import jax.numpy as jnp
from jax import lax
from jax.experimental import pallas as pl
from jax.experimental.pallas import tpu as pltpu
from jax.experimental.pallas import tpu_sc as plsc

B, L, V, D = 4096, 200, 100000, 128
DE = 8
DO = D + 2 * DE
N_TOK = B * L

NC, NS = 2, 16
NW = NC * NS
TOK_PER_W = N_TOK // NW
K = 4
CHUNK = K * 128
N_CHUNKS = TOK_PER_W // CHUNK


def _body(x_hbm, xe_hbm, xn_hbm, w_hbm, wen_hbm, out_hbm,
          idx_v, e_v, n_v, combo_v, word_v, en_v, sem_w, sem_e):
    wid = lax.axis_index("s") * NC + lax.axis_index("c")

    def chunk_body(i, carry):
        base = wid * TOK_PER_W + i * CHUNK
        row = wid * (TOK_PER_W // 128) + i * K
        pltpu.sync_copy(x_hbm.at[pl.ds(row, K)], idx_v)
        pltpu.sync_copy(xe_hbm.at[pl.ds(row, K)], e_v)
        pltpu.sync_copy(xn_hbm.at[pl.ds(row, K)], n_v)
        for j in range(K):
            for t in range(8):
                sl = pl.ds(t * 16, 16)
                combo_v[j, sl] = e_v[j, sl] * 3 + n_v[j, sl]
        cps = []
        for j in range(K):
            cps.append(pltpu.async_copy(
                w_hbm.at[idx_v.at[j]], word_v.at[pl.ds(j * 128, 128)], sem_w))
            cps.append(pltpu.async_copy(
                wen_hbm.at[combo_v.at[j]], en_v.at[pl.ds(j * 128, 128)], sem_e))
        for c in cps:
            c.wait()
        pltpu.sync_copy(word_v, out_hbm.at[pl.ds(base, CHUNK), pl.ds(0, D)])
        pltpu.sync_copy(en_v, out_hbm.at[pl.ds(base, CHUNK), pl.ds(D, 2 * DE)])
        return carry

    lax.fori_loop(0, N_CHUNKS, chunk_body, 0)


@jax.jit
def _run(x_flat, xe_flat, xn_flat, w, w_en):
    mesh = plsc.VectorSubcoreMesh(core_axis_name="c", subcore_axis_name="s")
    f = pl.kernel(
        _body,
        out_type=jax.ShapeDtypeStruct((N_TOK, DO), jnp.float32),
        mesh=mesh,
        scratch_types=[
            pltpu.VMEM((K, 128), jnp.int32),
            pltpu.VMEM((K, 128), jnp.int32),
            pltpu.VMEM((K, 128), jnp.int32),
            pltpu.VMEM((K, 128), jnp.int32),
            pltpu.VMEM((CHUNK, D), jnp.float32),
            pltpu.VMEM((CHUNK, 2 * DE), jnp.float32),
            pltpu.SemaphoreType.DMA,
            pltpu.SemaphoreType.DMA,
        ],
        compiler_params=pltpu.CompilerParams(use_tc_tiling_on_sc=False),
    )
    return f(x_flat, xe_flat, xn_flat, w, w_en)


def kernel(x, x_entity, x_negation, target, text_inputs, use_elmo,
           W, W_entity, W_negation):
    w_en = jnp.concatenate(
        [jnp.repeat(W_entity, 3, axis=0), jnp.tile(W_negation, (3, 1))], axis=1)
    out = _run(x.reshape(N_TOK // 128, 128).astype(jnp.int32),
               x_entity.reshape(N_TOK // 128, 128).astype(jnp.int32),
               x_negation.reshape(N_TOK // 128, 128).astype(jnp.int32),
               W, w_en)
    return out.reshape(B, L, DO)
